# Initial kernel scaffold; baseline (speedup 1.0000x reference)
import jax
import jax.numpy as jnp
from jax.experimental import pallas as pl


def kernel(x, edge_index, batch, W1, b1, W2, b2, Wl, bl):
    raise NotImplementedError("write your pallas kernel here")


# speedup vs baseline: 5.1515x; 5.1515x over previous
import functools

import jax
import jax.numpy as jnp
from jax import lax
from jax.experimental import pallas as pl
from jax.experimental.pallas import tpu as pltpu
from jax.experimental.pallas import tpu_sc as plsc

N = 100000
E = 1600000
H = 64
NG = 128

NPAD = 102400
HALF = NPAD // 2
TRASH = 1024
ACC_R = HALF + TRASH
WB_R = HALF // 16

EP = 1605632
ROWS2D = EP // 128
SUB_ROWS = ROWS2D // 16
KB = 56
NKB = SUB_ROWS // KB
ZROWS = 200

BLK = 512
NBLK = NPAD // BLK
BLKP = 1000
NPB = N // BLKP

_f32 = jnp.float32


def _sc_mesh():
    return plsc.VectorSubcoreMesh(core_axis_name="c", subcore_axis_name="s")


ZROWS2 = 800


def _zero_acc(acc, zv, sid):
    for z in range(WB_R // ZROWS2):
        pltpu.sync_copy(zv, acc.at[pl.ds(sid * WB_R + z * ZROWS2, ZROWS2)])


def _write_back(acc, wb, out_hbm, base, sid):
    for z in range(WB_R // ZROWS2):
        pltpu.sync_copy(acc.at[pl.ds(sid * WB_R + z * ZROWS2, ZROWS2)], wb)
        pltpu.sync_copy(wb, out_hbm.at[pl.ds(base + sid * WB_R + z * ZROWS2,
                                             ZROWS2)])


@jax.jit
def _sc_count(dstloc, ones_hbm, zeros_hbm):
    @functools.partial(
        pl.kernel,
        mesh=_sc_mesh(),
        compiler_params=pltpu.CompilerParams(use_tc_tiling_on_sc=False),
        out_type=jax.ShapeDtypeStruct((NPAD, 16), _f32),
        scratch_types=[
            pltpu.VMEM((128,), jnp.int32),
            pltpu.VMEM((128, 16), _f32),
            pltpu.VMEM((ZROWS2, 16), _f32),
            pltpu.VMEM((ZROWS2, 16), _f32),
            pltpu.VMEM_SHARED((ACC_R, 16), _f32),
            pltpu.SemaphoreType.DMA,
        ],
    )
    def k(dst_hbm, ones_h, zeros_h, out_hbm, idxd, ones_v, zv, wb, acc, sem):
        cid = lax.axis_index("c")
        sid = lax.axis_index("s")
        pltpu.sync_copy(ones_h, ones_v)
        pltpu.sync_copy(zeros_h, zv)
        _zero_acc(acc, zv, sid)
        plsc.subcore_barrier()

        @pl.loop(0, SUB_ROWS)
        def _(r):
            row = cid * ROWS2D + sid * SUB_ROWS + r
            pltpu.async_copy(dst_hbm.at[row], idxd, sem).wait()
            pltpu.sync_copy(ones_v, acc.at[idxd], add=True)

        plsc.subcore_barrier()
        _write_back(acc, wb, out_hbm, cid * HALF, sid)

    return k(dstloc, ones_hbm, zeros_hbm)


def _sc_propagate(table, srcg, dstloc, zeros_hbm, npass):
    @functools.partial(
        pl.kernel,
        mesh=_sc_mesh(),
        compiler_params=pltpu.CompilerParams(use_tc_tiling_on_sc=False),
        out_type=jax.ShapeDtypeStruct((npass * NPAD, 16), _f32),
        scratch_types=[
            pltpu.VMEM((128,), jnp.int32),
            pltpu.VMEM((128,), jnp.int32),
            pltpu.VMEM((128, 16), _f32),
            pltpu.VMEM((ZROWS2, 16), _f32),
            pltpu.VMEM((ZROWS2, 16), _f32),
            pltpu.VMEM_SHARED((ACC_R, 16), _f32),
            pltpu.SemaphoreType.DMA,
        ],
    )
    def k(table_hbm, src_hbm, dst_hbm, zeros_h, out_hbm,
          idxs, idxd, rows, zv, wb, acc, sem):
        cid = lax.axis_index("c")
        sid = lax.axis_index("s")
        pltpu.sync_copy(zeros_h, zv)
        _zero_acc(acc, zv, sid)
        for c in range(npass):
            plsc.subcore_barrier()

            @pl.loop(0, SUB_ROWS)
            def _(r):
                row = sid * SUB_ROWS + r
                pltpu.async_copy(src_hbm.at[c * ROWS2D + row], idxs, sem).wait()
                pltpu.async_copy(
                    dst_hbm.at[cid * ROWS2D + row], idxd, sem).wait()
                pltpu.async_copy(table_hbm.at[idxs], rows, sem).wait()
                pltpu.sync_copy(rows, acc.at[idxd], add=True)

            plsc.subcore_barrier()
            _write_back(acc, wb, out_hbm, c * NPAD + cid * HALF, sid)
            if c + 1 < npass:
                _zero_acc(acc, zv, sid)

    return k(table, srcg, dstloc, zeros_hbm)


def _tc_prep(cntp, xp16):
    def body(c_ref, x_ref, dinv_ref, g0_ref):
        dinv = lax.rsqrt(c_ref[:, 0:1] + 1.0)
        dinv_ref[...] = dinv
        g0_ref[...] = dinv * x_ref[...]

    return pl.pallas_call(
        body,
        grid=(NBLK,),
        in_specs=[
            pl.BlockSpec((BLK, 16), lambda i: (i, 0)),
            pl.BlockSpec((BLK, 16), lambda i: (i, 0)),
        ],
        out_specs=[
            pl.BlockSpec((BLK, 1), lambda i: (i, 0)),
            pl.BlockSpec((BLK, 16), lambda i: (i, 0)),
        ],
        out_shape=[
            jax.ShapeDtypeStruct((NPAD, 1), _f32),
            jax.ShapeDtypeStruct((NPAD, 16), _f32),
        ],
    )(cntp, xp16)


def _tc_dense(p, g, dinv, W, b, scale_out):
    K = W.shape[0]

    def body(p_ref, g_ref, d_ref, w_ref, b_ref, o_ref):
        d = d_ref[...]
        pre = d * (p_ref[...] + g_ref[...])
        h = jnp.dot(pre, w_ref[...], preferred_element_type=_f32) + b_ref[...]
        h = jnp.maximum(h, 0.0)
        o_ref[...] = d * h if scale_out else h

    return pl.pallas_call(
        body,
        grid=(NBLK,),
        in_specs=[
            pl.BlockSpec((BLK, K), lambda i: (i, 0)),
            pl.BlockSpec((BLK, K), lambda i: (i, 0)),
            pl.BlockSpec((BLK, 1), lambda i: (i, 0)),
            pl.BlockSpec((K, 64), lambda i: (0, 0)),
            pl.BlockSpec((1, 64), lambda i: (0, 0)),
        ],
        out_specs=pl.BlockSpec((BLK, 64), lambda i: (i, 0)),
        out_shape=jax.ShapeDtypeStruct((NPAD, 64), _f32),
    )(p, g, dinv, W, b)


def _tc_pool(h2, batchr, Wlp, blp):
    def body(h_ref, b_ref, wl_ref, bl_ref, o_ref, sums, cnts):
        i = pl.program_id(0)
        bb = b_ref[0]
        onehot_t = (lax.broadcasted_iota(jnp.int32, (NG, BLKP), 0) == bb
                    ).astype(_f32)
        dn = (((1,), (0,)), ((), ()))
        s_inc = lax.dot_general(onehot_t, h_ref[...], dn,
                                preferred_element_type=_f32)
        c_inc = lax.dot_general(onehot_t, jnp.ones((BLKP, 8), _f32), dn,
                                preferred_element_type=_f32)

        @pl.when(i == 0)
        def _():
            sums[...] = jnp.zeros_like(sums)
            cnts[...] = jnp.zeros_like(cnts)

        sums[...] += s_inc
        cnts[...] += c_inc

        @pl.when(i == NPB - 1)
        def _():
            pooled = sums[...] / jnp.maximum(cnts[...][:, 0:1], 1.0)
            o_ref[...] = jnp.dot(pooled, wl_ref[...],
                                 preferred_element_type=_f32) + bl_ref[...]

    return pl.pallas_call(
        body,
        grid=(NPB,),
        in_specs=[
            pl.BlockSpec((BLKP, 64), lambda i: (i, 0)),
            pl.BlockSpec((1, 1, BLKP), lambda i: (i, 0, 0)),
            pl.BlockSpec((64, 8), lambda i: (0, 0)),
            pl.BlockSpec((1, 8), lambda i: (0, 0)),
        ],
        out_specs=pl.BlockSpec((NG, 8), lambda i: (0, 0)),
        out_shape=jax.ShapeDtypeStruct((NG, 8), _f32),
        scratch_shapes=[
            pltpu.VMEM((NG, 64), _f32),
            pltpu.VMEM((NG, 8), _f32),
        ],
    )(h2, batchr, Wlp, blp)


def kernel(x, edge_index, batch, W1, b1, W2, b2, Wl, bl):
    src = edge_index[0].astype(jnp.int32)
    dst = edge_index[1].astype(jnp.int32)
    pad = jnp.full((EP - E,), N, jnp.int32)
    src2d = jnp.concatenate([src, pad]).reshape(ROWS2D, 128)
    dst2d = jnp.concatenate([dst, pad]).reshape(ROWS2D, 128)
    dstloc = []
    for lo in (0, HALF):
        in_half = (dst2d >= lo) & (dst2d < lo + HALF)
        dstloc.append(jnp.where(in_half, dst2d - lo,
                                HALF + (dst2d & (TRASH - 1))))
    dstloc = jnp.concatenate(dstloc)
    ones_hbm = jnp.ones((128, 16), _f32)
    zeros_hbm = jnp.zeros((ZROWS2, 16), _f32)

    xp16 = jnp.concatenate(
        [x.astype(_f32), jnp.zeros((N, 12), _f32)], axis=1)
    xp16 = jnp.concatenate([xp16, jnp.zeros((NPAD - N, 16), _f32)], axis=0)

    cntp = _sc_count(dstloc, ones_hbm, zeros_hbm)
    dinv, g0 = _tc_prep(cntp, xp16)

    p1 = _sc_propagate(g0, src2d, dstloc, zeros_hbm, 1)
    W1p = jnp.concatenate([W1.astype(_f32), jnp.zeros((12, 64), _f32)], axis=0)
    g1 = _tc_dense(p1, g0, dinv, W1p, b1.reshape(1, 64), True)

    table2 = g1.reshape(NPAD, 4, 16).transpose(1, 0, 2).reshape(4 * NPAD, 16)
    srcg2 = jnp.concatenate([src2d + c * NPAD for c in range(4)])
    p2 = _sc_propagate(table2, srcg2, dstloc, zeros_hbm, 4)
    p2t = p2.reshape(4, NPAD, 16).transpose(1, 0, 2).reshape(NPAD, 64)
    h2 = _tc_dense(p2t, g1, dinv, W2.astype(_f32), b2.reshape(1, 64), False)

    batchr = batch.astype(jnp.int32).reshape(NPB, 1, BLKP)
    Wlp = jnp.concatenate([Wl.astype(_f32), jnp.zeros((64, 7), _f32)], axis=1)
    blp = jnp.concatenate([bl.astype(_f32), jnp.zeros((7,), _f32)]).reshape(1, 8)
    out8 = _tc_pool(h2[:N], batchr, Wlp, blp)
    return out8[:, 0:1]


# speedup vs baseline: 10.6606x; 2.0694x over previous
import functools

import jax
import jax.numpy as jnp
from jax import lax
from jax.experimental import pallas as pl
from jax.experimental.pallas import tpu as pltpu
from jax.experimental.pallas import tpu_sc as plsc

N = 100000
E = 1600000
H = 64
NG = 128

NPAD = 102400
HALF = NPAD // 2
TRASH = 1024
ACC_R = HALF + TRASH
WB_R = HALF // 16

EP = 1605632
ROWS2D = EP // 128
SUB_ROWS = ROWS2D // 16
KB = 56
NKB = SUB_ROWS // KB
ZROWS = 200

BLK = 512
NBLK = NPAD // BLK
BLKP = 1000
NPB = N // BLKP

_f32 = jnp.float32


def _sc_mesh():
    return plsc.VectorSubcoreMesh(core_axis_name="c", subcore_axis_name="s")


ZROWS2 = 800


def _zero_acc(acc, zv, sid):
    for z in range(WB_R // ZROWS2):
        pltpu.sync_copy(zv, acc.at[pl.ds(sid * WB_R + z * ZROWS2, ZROWS2)])


def _write_back(acc, wb, out_hbm, base, sid):
    for z in range(WB_R // ZROWS2):
        pltpu.sync_copy(acc.at[pl.ds(sid * WB_R + z * ZROWS2, ZROWS2)], wb)
        pltpu.sync_copy(wb, out_hbm.at[pl.ds(base + sid * WB_R + z * ZROWS2,
                                             ZROWS2)])


@jax.jit
def _sc_count(dstloc, ones_hbm, zeros_hbm):
    @functools.partial(
        pl.kernel,
        mesh=_sc_mesh(),
        compiler_params=pltpu.CompilerParams(use_tc_tiling_on_sc=False),
        out_type=jax.ShapeDtypeStruct((NPAD, 16), _f32),
        scratch_types=[
            pltpu.VMEM((128,), jnp.int32),
            pltpu.VMEM((128,), jnp.int32),
            pltpu.VMEM((128, 16), _f32),
            pltpu.VMEM((ZROWS2, 16), _f32),
            pltpu.VMEM((ZROWS2, 16), _f32),
            pltpu.VMEM_SHARED((ACC_R, 16), _f32),
            pltpu.SemaphoreType.DMA,
            pltpu.SemaphoreType.DMA,
        ],
    )
    def k(dst_hbm, ones_h, zeros_h, out_hbm, idxd, idxd1, ones_v, zv, wb, acc,
          sem, sem1):
        cid = lax.axis_index("c")
        sid = lax.axis_index("s")
        pltpu.sync_copy(ones_h, ones_v)
        pltpu.sync_copy(zeros_h, zv)
        _zero_acc(acc, zv, sid)
        plsc.subcore_barrier()

        def start_idx(r, idxd_b, sem_b):
            rr = jnp.minimum(r, SUB_ROWS - 1)
            pltpu.async_copy(
                dst_hbm.at[cid * ROWS2D + sid * SUB_ROWS + rr], idxd_b, sem_b)

        def wait_idx(idxd_b, sem_b):
            pltpu.make_async_copy(dst_hbm.at[0], idxd_b, sem_b).wait()

        start_idx(0, idxd, sem)
        start_idx(1, idxd1, sem1)

        @pl.loop(0, SUB_ROWS // 2)
        def _(t):
            r0 = 2 * t
            wait_idx(idxd, sem)
            pltpu.sync_copy(ones_v, acc.at[idxd], add=True)
            start_idx(r0 + 2, idxd, sem)
            wait_idx(idxd1, sem1)
            pltpu.sync_copy(ones_v, acc.at[idxd1], add=True)
            start_idx(r0 + 3, idxd1, sem1)

        wait_idx(idxd, sem)
        wait_idx(idxd1, sem1)
        plsc.subcore_barrier()
        _write_back(acc, wb, out_hbm, cid * HALF, sid)

    return k(dstloc, ones_hbm, zeros_hbm)


def _sc_propagate(table, srcg, dstloc, zeros_hbm, npass):
    @functools.partial(
        pl.kernel,
        mesh=_sc_mesh(),
        compiler_params=pltpu.CompilerParams(use_tc_tiling_on_sc=False),
        out_type=jax.ShapeDtypeStruct((npass * NPAD, 16), _f32),
        scratch_types=[
            pltpu.VMEM((128,), jnp.int32),
            pltpu.VMEM((128,), jnp.int32),
            pltpu.VMEM((128,), jnp.int32),
            pltpu.VMEM((128,), jnp.int32),
            pltpu.VMEM((128, 16), _f32),
            pltpu.VMEM((128, 16), _f32),
            pltpu.VMEM((ZROWS2, 16), _f32),
            pltpu.VMEM((ZROWS2, 16), _f32),
            pltpu.VMEM_SHARED((ACC_R, 16), _f32),
            pltpu.SemaphoreType.DMA,
            pltpu.SemaphoreType.DMA,
            pltpu.SemaphoreType.DMA,
            pltpu.SemaphoreType.DMA,
        ],
    )
    def k(table_hbm, src_hbm, dst_hbm, zeros_h, out_hbm,
          idxs0, idxs1, idxd0, idxd1, rows0, rows1, zv, wb, acc,
          semi0, semi1, semg0, semg1):
        cid = lax.axis_index("c")
        sid = lax.axis_index("s")
        pltpu.sync_copy(zeros_h, zv)
        _zero_acc(acc, zv, sid)

        def start_idx(c, r, idxs_b, idxd_b, sem_b):
            rr = jnp.minimum(r, SUB_ROWS - 1)
            row = sid * SUB_ROWS + rr
            pltpu.async_copy(src_hbm.at[c * ROWS2D + row], idxs_b, sem_b)
            pltpu.async_copy(dst_hbm.at[cid * ROWS2D + row], idxd_b, sem_b)

        def wait_idx(idxs_b, idxd_b, sem_b):
            pltpu.make_async_copy(src_hbm.at[0], idxs_b, sem_b).wait()
            pltpu.make_async_copy(src_hbm.at[0], idxd_b, sem_b).wait()

        for c in range(npass):
            plsc.subcore_barrier()
            start_idx(c, 0, idxs0, idxd0, semi0)
            start_idx(c, 1, idxs1, idxd1, semi1)

            @pl.loop(0, SUB_ROWS // 2)
            def _(t):
                r0 = 2 * t
                wait_idx(idxs0, idxd0, semi0)
                g0 = pltpu.async_copy(table_hbm.at[idxs0], rows0, semg0)
                wait_idx(idxs1, idxd1, semi1)
                g1 = pltpu.async_copy(table_hbm.at[idxs1], rows1, semg1)
                g0.wait()
                pltpu.sync_copy(rows0, acc.at[idxd0], add=True)
                start_idx(c, r0 + 2, idxs0, idxd0, semi0)
                g1.wait()
                pltpu.sync_copy(rows1, acc.at[idxd1], add=True)
                start_idx(c, r0 + 3, idxs1, idxd1, semi1)

            wait_idx(idxs0, idxd0, semi0)
            wait_idx(idxs1, idxd1, semi1)
            plsc.subcore_barrier()
            _write_back(acc, wb, out_hbm, c * NPAD + cid * HALF, sid)
            if c + 1 < npass:
                _zero_acc(acc, zv, sid)

    return k(table, srcg, dstloc, zeros_hbm)


def _tc_prep(cntp, xp16):
    def body(c_ref, x_ref, dinv_ref, g0_ref):
        dinv = lax.rsqrt(c_ref[:, 0:1] + 1.0)
        dinv_ref[...] = dinv
        g0_ref[...] = dinv * x_ref[...]

    return pl.pallas_call(
        body,
        grid=(NBLK,),
        in_specs=[
            pl.BlockSpec((BLK, 16), lambda i: (i, 0)),
            pl.BlockSpec((BLK, 16), lambda i: (i, 0)),
        ],
        out_specs=[
            pl.BlockSpec((BLK, 1), lambda i: (i, 0)),
            pl.BlockSpec((BLK, 16), lambda i: (i, 0)),
        ],
        out_shape=[
            jax.ShapeDtypeStruct((NPAD, 1), _f32),
            jax.ShapeDtypeStruct((NPAD, 16), _f32),
        ],
    )(cntp, xp16)


def _tc_dense(p, g, dinv, W, b, scale_out):
    K = W.shape[0]

    def body(p_ref, g_ref, d_ref, w_ref, b_ref, o_ref):
        d = d_ref[...]
        pre = d * (p_ref[...] + g_ref[...])
        h = jnp.dot(pre, w_ref[...], preferred_element_type=_f32) + b_ref[...]
        h = jnp.maximum(h, 0.0)
        o_ref[...] = d * h if scale_out else h

    return pl.pallas_call(
        body,
        grid=(NBLK,),
        in_specs=[
            pl.BlockSpec((BLK, K), lambda i: (i, 0)),
            pl.BlockSpec((BLK, K), lambda i: (i, 0)),
            pl.BlockSpec((BLK, 1), lambda i: (i, 0)),
            pl.BlockSpec((K, 64), lambda i: (0, 0)),
            pl.BlockSpec((1, 64), lambda i: (0, 0)),
        ],
        out_specs=pl.BlockSpec((BLK, 64), lambda i: (i, 0)),
        out_shape=jax.ShapeDtypeStruct((NPAD, 64), _f32),
    )(p, g, dinv, W, b)


def _tc_pool(h2, batchr, Wlp, blp):
    def body(h_ref, b_ref, wl_ref, bl_ref, o_ref, sums, cnts):
        i = pl.program_id(0)
        bb = b_ref[0]
        onehot_t = (lax.broadcasted_iota(jnp.int32, (NG, BLKP), 0) == bb
                    ).astype(_f32)
        dn = (((1,), (0,)), ((), ()))
        s_inc = lax.dot_general(onehot_t, h_ref[...], dn,
                                preferred_element_type=_f32)
        c_inc = lax.dot_general(onehot_t, jnp.ones((BLKP, 8), _f32), dn,
                                preferred_element_type=_f32)

        @pl.when(i == 0)
        def _():
            sums[...] = jnp.zeros_like(sums)
            cnts[...] = jnp.zeros_like(cnts)

        sums[...] += s_inc
        cnts[...] += c_inc

        @pl.when(i == NPB - 1)
        def _():
            pooled = sums[...] / jnp.maximum(cnts[...][:, 0:1], 1.0)
            o_ref[...] = jnp.dot(pooled, wl_ref[...],
                                 preferred_element_type=_f32) + bl_ref[...]

    return pl.pallas_call(
        body,
        grid=(NPB,),
        in_specs=[
            pl.BlockSpec((BLKP, 64), lambda i: (i, 0)),
            pl.BlockSpec((1, 1, BLKP), lambda i: (i, 0, 0)),
            pl.BlockSpec((64, 8), lambda i: (0, 0)),
            pl.BlockSpec((1, 8), lambda i: (0, 0)),
        ],
        out_specs=pl.BlockSpec((NG, 8), lambda i: (0, 0)),
        out_shape=jax.ShapeDtypeStruct((NG, 8), _f32),
        scratch_shapes=[
            pltpu.VMEM((NG, 64), _f32),
            pltpu.VMEM((NG, 8), _f32),
        ],
    )(h2, batchr, Wlp, blp)


def kernel(x, edge_index, batch, W1, b1, W2, b2, Wl, bl):
    src = edge_index[0].astype(jnp.int32)
    dst = edge_index[1].astype(jnp.int32)
    pad = jnp.full((EP - E,), N, jnp.int32)
    src2d = jnp.concatenate([src, pad]).reshape(ROWS2D, 128)
    dst2d = jnp.concatenate([dst, pad]).reshape(ROWS2D, 128)
    dstloc = []
    for lo in (0, HALF):
        in_half = (dst2d >= lo) & (dst2d < lo + HALF)
        dstloc.append(jnp.where(in_half, dst2d - lo,
                                HALF + (dst2d & (TRASH - 1))))
    dstloc = jnp.concatenate(dstloc)
    ones_hbm = jnp.ones((128, 16), _f32)
    zeros_hbm = jnp.zeros((ZROWS2, 16), _f32)

    xp16 = jnp.concatenate(
        [x.astype(_f32), jnp.zeros((N, 12), _f32)], axis=1)
    xp16 = jnp.concatenate([xp16, jnp.zeros((NPAD - N, 16), _f32)], axis=0)

    cntp = _sc_count(dstloc, ones_hbm, zeros_hbm)
    dinv, g0 = _tc_prep(cntp, xp16)

    p1 = _sc_propagate(g0, src2d, dstloc, zeros_hbm, 1)
    W1p = jnp.concatenate([W1.astype(_f32), jnp.zeros((12, 64), _f32)], axis=0)
    g1 = _tc_dense(p1, g0, dinv, W1p, b1.reshape(1, 64), True)

    table2 = g1.reshape(NPAD, 4, 16).transpose(1, 0, 2).reshape(4 * NPAD, 16)
    srcg2 = jnp.concatenate([src2d + c * NPAD for c in range(4)])
    p2 = _sc_propagate(table2, srcg2, dstloc, zeros_hbm, 4)
    p2t = p2.reshape(4, NPAD, 16).transpose(1, 0, 2).reshape(NPAD, 64)
    h2 = _tc_dense(p2t, g1, dinv, W2.astype(_f32), b2.reshape(1, 64), False)

    batchr = batch.astype(jnp.int32).reshape(NPB, 1, BLKP)
    Wlp = jnp.concatenate([Wl.astype(_f32), jnp.zeros((64, 7), _f32)], axis=1)
    blp = jnp.concatenate([bl.astype(_f32), jnp.zeros((7,), _f32)]).reshape(1, 8)
    out8 = _tc_pool(h2[:N], batchr, Wlp, blp)
    return out8[:, 0:1]


# speedup vs baseline: 14.6799x; 1.3770x over previous
import functools

import jax
import jax.numpy as jnp
from jax import lax
from jax.experimental import pallas as pl
from jax.experimental.pallas import tpu as pltpu
from jax.experimental.pallas import tpu_sc as plsc

N = 100000
E = 1600000
H = 64
NG = 128

NPAD = 102400
HALF = NPAD // 2
TRASH = 1024
ACC_R = HALF + TRASH
WB_R = HALF // 16

EP = 1605632
ROWS2D = EP // 128
SUB_ROWS = ROWS2D // 16
KB = 56
NKB = SUB_ROWS // KB
ZROWS = 200

BLK = 512
NBLK = NPAD // BLK
BLKP = 1000
NPB = N // BLKP

_f32 = jnp.float32


def _sc_mesh():
    return plsc.VectorSubcoreMesh(core_axis_name="c", subcore_axis_name="s")


ZROWS2 = 800


def _zero_acc(acc, zv, sid):
    for z in range(WB_R // ZROWS2):
        pltpu.sync_copy(zv, acc.at[pl.ds(sid * WB_R + z * ZROWS2, ZROWS2)])


def _write_back(acc, wb, out_hbm, base, sid):
    for z in range(WB_R // ZROWS2):
        pltpu.sync_copy(acc.at[pl.ds(sid * WB_R + z * ZROWS2, ZROWS2)], wb)
        pltpu.sync_copy(wb, out_hbm.at[pl.ds(base + sid * WB_R + z * ZROWS2,
                                             ZROWS2)])


@jax.jit
def _sc_count(dstloc, ones_hbm, zeros_hbm):
    @functools.partial(
        pl.kernel,
        mesh=_sc_mesh(),
        compiler_params=pltpu.CompilerParams(use_tc_tiling_on_sc=False),
        out_type=jax.ShapeDtypeStruct((NPAD, 16), _f32),
        scratch_types=[
            pltpu.VMEM((128,), jnp.int32),
            pltpu.VMEM((128,), jnp.int32),
            pltpu.VMEM((128, 16), _f32),
            pltpu.VMEM((ZROWS2, 16), _f32),
            pltpu.VMEM((ZROWS2, 16), _f32),
            pltpu.VMEM_SHARED((ACC_R, 16), _f32),
            pltpu.SemaphoreType.DMA,
            pltpu.SemaphoreType.DMA,
        ],
    )
    def k(dst_hbm, ones_h, zeros_h, out_hbm, idxd, idxd1, ones_v, zv, wb, acc,
          sem, sem1):
        cid = lax.axis_index("c")
        sid = lax.axis_index("s")
        pltpu.sync_copy(ones_h, ones_v)
        pltpu.sync_copy(zeros_h, zv)
        _zero_acc(acc, zv, sid)
        plsc.subcore_barrier()

        def start_idx(r, idxd_b, sem_b):
            rr = jnp.minimum(r, SUB_ROWS - 1)
            pltpu.async_copy(
                dst_hbm.at[cid * ROWS2D + sid * SUB_ROWS + rr], idxd_b, sem_b)

        def wait_idx(idxd_b, sem_b):
            pltpu.make_async_copy(dst_hbm.at[0], idxd_b, sem_b).wait()

        start_idx(0, idxd, sem)
        start_idx(1, idxd1, sem1)

        @pl.loop(0, SUB_ROWS // 2)
        def _(t):
            r0 = 2 * t
            wait_idx(idxd, sem)
            pltpu.sync_copy(ones_v, acc.at[idxd], add=True)
            start_idx(r0 + 2, idxd, sem)
            wait_idx(idxd1, sem1)
            pltpu.sync_copy(ones_v, acc.at[idxd1], add=True)
            start_idx(r0 + 3, idxd1, sem1)

        wait_idx(idxd, sem)
        wait_idx(idxd1, sem1)
        plsc.subcore_barrier()
        _write_back(acc, wb, out_hbm, cid * HALF, sid)

    return k(dstloc, ones_hbm, zeros_hbm)


def _sc_propagate(table, srcg, dstloc, zeros_hbm, npass):
    @functools.partial(
        pl.kernel,
        mesh=_sc_mesh(),
        compiler_params=pltpu.CompilerParams(use_tc_tiling_on_sc=False),
        out_type=jax.ShapeDtypeStruct((npass * NPAD, 16), _f32),
        scratch_types=(
            [pltpu.VMEM((128,), jnp.int32)] * 8
            + [pltpu.VMEM((128, 16), _f32)] * 4
            + [pltpu.VMEM((ZROWS2, 16), _f32)] * 2
            + [pltpu.VMEM_SHARED((ACC_R, 16), _f32)]
            + [pltpu.SemaphoreType.DMA] * 8
        ),
    )
    def k(table_hbm, src_hbm, dst_hbm, zeros_h, out_hbm,
          i_s0, i_s1, i_s2, i_s3, i_d0, i_d1, i_d2, i_d3,
          rw0, rw1, rw2, rw3, zv, wb, acc,
          mi0, mi1, mi2, mi3, mg0, mg1, mg2, mg3):
        cid = lax.axis_index("c")
        sid = lax.axis_index("s")
        pltpu.sync_copy(zeros_h, zv)
        _zero_acc(acc, zv, sid)
        isets = ((i_s0, i_d0, mi0, rw0, mg0), (i_s1, i_d1, mi1, rw1, mg1),
                 (i_s2, i_d2, mi2, rw2, mg2), (i_s3, i_d3, mi3, rw3, mg3))

        def start_idx(c, r, bset):
            idxs_b, idxd_b, sem_b = bset[0], bset[1], bset[2]
            rr = jnp.minimum(r, SUB_ROWS - 1)
            row = sid * SUB_ROWS + rr
            pltpu.async_copy(src_hbm.at[c * ROWS2D + row], idxs_b, sem_b)
            pltpu.async_copy(dst_hbm.at[cid * ROWS2D + row], idxd_b, sem_b)

        def wait_idx(bset):
            idxs_b, idxd_b, sem_b = bset[0], bset[1], bset[2]
            pltpu.make_async_copy(src_hbm.at[0], idxs_b, sem_b).wait()
            pltpu.make_async_copy(src_hbm.at[0], idxd_b, sem_b).wait()

        for c in range(npass):
            plsc.subcore_barrier()
            for b in range(4):
                start_idx(c, b, isets[b])

            @pl.loop(0, SUB_ROWS // 4)
            def _(t):
                r0 = 4 * t
                gs = []
                for b in range(4):
                    wait_idx(isets[b])
                    gs.append(pltpu.async_copy(
                        table_hbm.at[isets[b][0]], isets[b][3], isets[b][4]))
                for b in range(4):
                    gs[b].wait()
                    pltpu.sync_copy(isets[b][3], acc.at[isets[b][1]], add=True)
                    start_idx(c, r0 + 4 + b, isets[b])

            for b in range(4):
                wait_idx(isets[b])
            plsc.subcore_barrier()
            _write_back(acc, wb, out_hbm, c * NPAD + cid * HALF, sid)
            if c + 1 < npass:
                _zero_acc(acc, zv, sid)

    return k(table, srcg, dstloc, zeros_hbm)


def _tc_prep(cntp, xp16):
    def body(c_ref, x_ref, dinv_ref, g0_ref):
        dinv = lax.rsqrt(c_ref[:, 0:1] + 1.0)
        dinv_ref[...] = dinv
        g0_ref[...] = dinv * x_ref[...]

    return pl.pallas_call(
        body,
        grid=(NBLK,),
        in_specs=[
            pl.BlockSpec((BLK, 16), lambda i: (i, 0)),
            pl.BlockSpec((BLK, 16), lambda i: (i, 0)),
        ],
        out_specs=[
            pl.BlockSpec((BLK, 1), lambda i: (i, 0)),
            pl.BlockSpec((BLK, 16), lambda i: (i, 0)),
        ],
        out_shape=[
            jax.ShapeDtypeStruct((NPAD, 1), _f32),
            jax.ShapeDtypeStruct((NPAD, 16), _f32),
        ],
    )(cntp, xp16)


def _tc_dense(p, g, dinv, W, b, scale_out):
    K = W.shape[0]

    def body(p_ref, g_ref, d_ref, w_ref, b_ref, o_ref):
        d = d_ref[...]
        pre = d * (p_ref[...] + g_ref[...])
        h = jnp.dot(pre, w_ref[...], preferred_element_type=_f32) + b_ref[...]
        h = jnp.maximum(h, 0.0)
        o_ref[...] = d * h if scale_out else h

    return pl.pallas_call(
        body,
        grid=(NBLK,),
        in_specs=[
            pl.BlockSpec((BLK, K), lambda i: (i, 0)),
            pl.BlockSpec((BLK, K), lambda i: (i, 0)),
            pl.BlockSpec((BLK, 1), lambda i: (i, 0)),
            pl.BlockSpec((K, 64), lambda i: (0, 0)),
            pl.BlockSpec((1, 64), lambda i: (0, 0)),
        ],
        out_specs=pl.BlockSpec((BLK, 64), lambda i: (i, 0)),
        out_shape=jax.ShapeDtypeStruct((NPAD, 64), _f32),
    )(p, g, dinv, W, b)


def _tc_pool(h2, batchr, Wlp, blp):
    def body(h_ref, b_ref, wl_ref, bl_ref, o_ref, sums, cnts):
        i = pl.program_id(0)
        bb = b_ref[0]
        onehot_t = (lax.broadcasted_iota(jnp.int32, (NG, BLKP), 0) == bb
                    ).astype(_f32)
        dn = (((1,), (0,)), ((), ()))
        s_inc = lax.dot_general(onehot_t, h_ref[...], dn,
                                preferred_element_type=_f32)
        c_inc = lax.dot_general(onehot_t, jnp.ones((BLKP, 8), _f32), dn,
                                preferred_element_type=_f32)

        @pl.when(i == 0)
        def _():
            sums[...] = jnp.zeros_like(sums)
            cnts[...] = jnp.zeros_like(cnts)

        sums[...] += s_inc
        cnts[...] += c_inc

        @pl.when(i == NPB - 1)
        def _():
            pooled = sums[...] / jnp.maximum(cnts[...][:, 0:1], 1.0)
            o_ref[...] = jnp.dot(pooled, wl_ref[...],
                                 preferred_element_type=_f32) + bl_ref[...]

    return pl.pallas_call(
        body,
        grid=(NPB,),
        in_specs=[
            pl.BlockSpec((BLKP, 64), lambda i: (i, 0)),
            pl.BlockSpec((1, 1, BLKP), lambda i: (i, 0, 0)),
            pl.BlockSpec((64, 8), lambda i: (0, 0)),
            pl.BlockSpec((1, 8), lambda i: (0, 0)),
        ],
        out_specs=pl.BlockSpec((NG, 8), lambda i: (0, 0)),
        out_shape=jax.ShapeDtypeStruct((NG, 8), _f32),
        scratch_shapes=[
            pltpu.VMEM((NG, 64), _f32),
            pltpu.VMEM((NG, 8), _f32),
        ],
    )(h2, batchr, Wlp, blp)


def kernel(x, edge_index, batch, W1, b1, W2, b2, Wl, bl):
    src = edge_index[0].astype(jnp.int32)
    dst = edge_index[1].astype(jnp.int32)
    pad = jnp.full((EP - E,), N, jnp.int32)
    src2d = jnp.concatenate([src, pad]).reshape(ROWS2D, 128)
    dst2d = jnp.concatenate([dst, pad]).reshape(ROWS2D, 128)
    dstloc = []
    for lo in (0, HALF):
        in_half = (dst2d >= lo) & (dst2d < lo + HALF)
        dstloc.append(jnp.where(in_half, dst2d - lo,
                                HALF + (dst2d & (TRASH - 1))))
    dstloc = jnp.concatenate(dstloc)
    ones_hbm = jnp.ones((128, 16), _f32)
    zeros_hbm = jnp.zeros((ZROWS2, 16), _f32)

    xp16 = jnp.concatenate(
        [x.astype(_f32), jnp.zeros((N, 12), _f32)], axis=1)
    xp16 = jnp.concatenate([xp16, jnp.zeros((NPAD - N, 16), _f32)], axis=0)

    cntp = _sc_count(dstloc, ones_hbm, zeros_hbm)
    dinv, g0 = _tc_prep(cntp, xp16)

    p1 = _sc_propagate(g0, src2d, dstloc, zeros_hbm, 1)
    W1p = jnp.concatenate([W1.astype(_f32), jnp.zeros((12, 64), _f32)], axis=0)
    g1 = _tc_dense(p1, g0, dinv, W1p, b1.reshape(1, 64), True)

    table2 = g1.reshape(NPAD, 4, 16).transpose(1, 0, 2).reshape(4 * NPAD, 16)
    srcg2 = jnp.concatenate([src2d + c * NPAD for c in range(4)])
    p2 = _sc_propagate(table2, srcg2, dstloc, zeros_hbm, 4)
    p2t = p2.reshape(4, NPAD, 16).transpose(1, 0, 2).reshape(NPAD, 64)
    h2 = _tc_dense(p2t, g1, dinv, W2.astype(_f32), b2.reshape(1, 64), False)

    batchr = batch.astype(jnp.int32).reshape(NPB, 1, BLKP)
    Wlp = jnp.concatenate([Wl.astype(_f32), jnp.zeros((64, 7), _f32)], axis=1)
    blp = jnp.concatenate([bl.astype(_f32), jnp.zeros((7,), _f32)]).reshape(1, 8)
    out8 = _tc_pool(h2[:N], batchr, Wlp, blp)
    return out8[:, 0:1]
